# baseline (speedup 1.0000x reference)
import functools

import jax
import jax.numpy as jnp
import numpy as np
from jax import lax
from jax.experimental import pallas as pl
from jax.experimental.pallas import tpu as pltpu
from jax.experimental.pallas import tpu_sc as plsc

_N = 10000
_E = 320000
_D = 128
_H = 8
_C = _D // _H

_NODE_BLK = 1000
_EDGE_BLK = 4000

_NW = 32
_EPT = _E // _NW
_EB = 64
_EPT_MAIN = 10240
_NBLK_MAIN = _EPT_MAIN // _EB
_NPT = 640
_N_PAD = 16 * _NPT
_DPT = _NPT // 16


def _node_proj_body(x_ref, g1_ref, b1_ref, wq_ref, bq_ref, wk_ref, bk_ref,
                    wv_ref, bv_ref, ws_ref, bs_ref,
                    q_ref, k_ref, v_ref, skip_ref):
    x = x_ref[...]
    mu = jnp.mean(x, axis=-1, keepdims=True)
    xc = x - mu
    var = jnp.mean(xc * xc, axis=-1, keepdims=True)
    h = xc * jax.lax.rsqrt(var + 1e-5) * g1_ref[...] + b1_ref[...]
    q = jnp.dot(h, wq_ref[...], preferred_element_type=jnp.float32) + bq_ref[...]
    q_ref[...] = q * (1.0 / np.sqrt(_C))
    k_ref[...] = jnp.dot(h, wk_ref[...], preferred_element_type=jnp.float32) + bk_ref[...]
    v_ref[...] = jnp.dot(h, wv_ref[...], preferred_element_type=jnp.float32) + bv_ref[...]
    skip_ref[...] = jnp.dot(h, ws_ref[...], preferred_element_type=jnp.float32) + bs_ref[...]


def _node_proj(x, g1, b1, Wq, bq, Wk, bk, Wv, bv, Wskip, bskip):
    grid = _N // _NODE_BLK
    row = pl.BlockSpec((_NODE_BLK, _D), lambda i: (i, 0))
    full = pl.BlockSpec((_D, _D), lambda i: (0, 0))
    vec = pl.BlockSpec((1, _D), lambda i: (0, 0))
    out_sd = jax.ShapeDtypeStruct((_N, _D), jnp.float32)
    return pl.pallas_call(
        _node_proj_body,
        grid=(grid,),
        in_specs=[row, vec, vec, full, vec, full, vec, full, vec, full, vec],
        out_specs=[row, row, row, row],
        out_shape=[out_sd, out_sd, out_sd, out_sd],
    )(x, g1.reshape(1, _D), b1.reshape(1, _D), Wq, bq.reshape(1, _D),
      Wk, bk.reshape(1, _D), Wv, bv.reshape(1, _D), Wskip, bskip.reshape(1, _D))


def _edge_proj_body(ea_ref, we_ref, be_ref, e_ref):
    e_ref[...] = jnp.dot(ea_ref[...], we_ref[...],
                         preferred_element_type=jnp.float32) + be_ref[...]


def _edge_proj(edge_attr, We, be):
    grid = _E // _EDGE_BLK
    row = pl.BlockSpec((_EDGE_BLK, _D), lambda i: (i, 0))
    return pl.pallas_call(
        _edge_proj_body,
        grid=(grid,),
        in_specs=[row, pl.BlockSpec((_D, _D), lambda i: (0, 0)),
                  pl.BlockSpec((1, _D), lambda i: (0, 0))],
        out_specs=row,
        out_shape=jax.ShapeDtypeStruct((_E, _D), jnp.float32),
    )(edge_attr, We, be.reshape(1, _D))


_OWN = _N_PAD // _NW
_SCAN = 512
_FB = 64


def _edge_contrib_sc(q, k, v, e, src, dst):
    mesh = plsc.VectorSubcoreMesh(core_axis_name="c", subcore_axis_name="s")

    @functools.partial(
        pl.kernel, mesh=mesh,
        out_type=[jax.ShapeDtypeStruct((_E + 8, _D), jnp.float32),
                  jax.ShapeDtypeStruct((_E // 8 + 8, _D), jnp.float32)],
        compiler_params=pltpu.CompilerParams(needs_layout_passes=False),
        scratch_types=[
            pltpu.VMEM((_EB,), jnp.int32),
            pltpu.VMEM((_EB,), jnp.int32),
            pltpu.VMEM((_EB, _D), jnp.float32),
            pltpu.VMEM((_EB, _D), jnp.float32),
            pltpu.VMEM((_EB, _D), jnp.float32),
            pltpu.VMEM((_EB, _D), jnp.float32),
            pltpu.VMEM((_EB, _D), jnp.float32),
            pltpu.VMEM((_EB // 8, _D), jnp.float32),
            pltpu.SemaphoreType.DMA,
            pltpu.SemaphoreType.DMA,
            pltpu.SemaphoreType.DMA,
            pltpu.SemaphoreType.DMA,
        ],
    )
    def body(q_hbm, k_hbm, v_hbm, e_hbm, src_hbm, dst_hbm, wv_hbm, wd_hbm,
             si, di, qr, kr, vr, er, orow, orowd, s0, s1, s2, s3):
        cid = lax.axis_index("c")
        sid = lax.axis_index("s")
        wid = sid * 2 + cid
        zeros16 = jnp.zeros((16,), jnp.float32)
        lanes = lax.iota(jnp.int32, 16)

        def zrow(r, c):
            for cb in range(_D // 16):
                orow[r, pl.ds(cb * 16, 16)] = zeros16
            return c
        lax.fori_loop(0, _EB, zrow, 0)

        def zrowd(r, c):
            for cb in range(_D // 16):
                orowd[r, pl.ds(cb * 16, 16)] = zeros16
            return c
        lax.fori_loop(0, _EB // 8, zrowd, 0)

        @pl.when(wid == 0)
        def _zpad():
            pltpu.sync_copy(orow.at[pl.ds(0, 8)], wv_hbm.at[pl.ds(_E, 8)])
            pltpu.sync_copy(orowd.at[pl.ds(0, 8)], wd_hbm.at[pl.ds(_E // 8, 8)])

        ebase = wid * _EPT_MAIN
        nblk = jnp.where(wid < 31, _NBLK_MAIN, (_E - 31 * _EPT_MAIN) // _EB)

        def eblk(j, c):
            off = pl.multiple_of(ebase + j * _EB, 8)
            pltpu.sync_copy(src_hbm.at[pl.ds(off, _EB)], si)
            pltpu.sync_copy(dst_hbm.at[pl.ds(off, _EB)], di)
            cq = pltpu.async_copy(q_hbm.at[di], qr, s0)
            ck = pltpu.async_copy(k_hbm.at[si], kr, s1)
            cv = pltpu.async_copy(v_hbm.at[si], vr, s2)
            ce = pltpu.async_copy(e_hbm.at[pl.ds(off, _EB)], er, s3)
            cq.wait()
            ck.wait()
            cv.wait()
            ce.wait()

            def edge(i, c2):
                dv = zeros16
                for h in range(_H):
                    sl = pl.ds(h * 16, 16)
                    ev = er[i, sl]
                    t = jnp.sum(qr[i, sl] * (kr[i, sl] + ev))
                    w = jnp.exp(jnp.full((16,), t, jnp.float32))
                    orow[i, sl] = w * (vr[i, sl] + ev)
                    dv = jnp.where(lanes == h, w, dv)
                orowd[lax.shift_right_logical(i, 3),
                      pl.ds((i & 7) * 16, 16)] = dv
                return c2
            lax.fori_loop(0, _EB, edge, 0, unroll=4)
            pltpu.sync_copy(orow, wv_hbm.at[pl.ds(off, _EB)])
            pltpu.sync_copy(
                orowd,
                wd_hbm.at[pl.ds(pl.multiple_of(
                    lax.shift_right_logical(off, 3), 8), _EB // 8)])
            return c
        lax.fori_loop(0, nblk, eblk, 0)

    return body(q, k, v, e, src, dst)


def _segment_reduce_sc(wv, wd, dst):
    mesh = plsc.VectorSubcoreMesh(core_axis_name="c", subcore_axis_name="s")

    @functools.partial(
        pl.kernel, mesh=mesh,
        out_type=[jax.ShapeDtypeStruct((_N_PAD, _D), jnp.float32),
                  jax.ShapeDtypeStruct((_N_PAD, 16), jnp.float32)],
        compiler_params=pltpu.CompilerParams(needs_layout_passes=False),
        scratch_types=[
            pltpu.VMEM((_OWN, _D), jnp.float32),
            pltpu.VMEM((_OWN, 16), jnp.float32),
            pltpu.VMEM((_SCAN,), jnp.int32),
            pltpu.VMEM((_FB,), jnp.int32),
            pltpu.VMEM((_FB,), jnp.int32),
            pltpu.VMEM((_FB,), jnp.int32),
            pltpu.VMEM((16,), jnp.int32),
            pltpu.VMEM((_FB, _D), jnp.float32),
            pltpu.VMEM((_FB, _D), jnp.float32),
            pltpu.SemaphoreType.DMA,
            pltpu.SemaphoreType.DMA,
        ],
    )
    def body(wv_hbm, wd_hbm, dst_hbm, outv_hbm, outd_hbm,
             accv, accd, dblk, idb, idbr, rowb, cur, wvg, wdg, s0, s1):
        cid = lax.axis_index("c")
        sid = lax.axis_index("s")
        wid = sid * 2 + cid
        zeros16 = jnp.zeros((16,), jnp.float32)
        zi16 = jnp.zeros((16,), jnp.int32)
        lanes = lax.iota(jnp.int32, 16)
        mybase = wid * _OWN
        dummy = jnp.full((16,), _E, jnp.int32)

        dummyr = jnp.full((16,), _E // 8, jnp.int32)

        def zacc(r, c):
            for cb in range(_D // 16):
                accv[r, pl.ds(cb * 16, 16)] = zeros16
            accd[r, pl.ds(0, 16)] = zeros16
            return c
        lax.fori_loop(0, _OWN, zacc, 0)
        for t in range(_FB // 16):
            idb[pl.ds(t * 16, 16)] = dummy
            idbr[pl.ds(t * 16, 16)] = dummyr
            rowb[pl.ds(t * 16, 16)] = zi16
        cur[pl.ds(0, 16)] = zi16

        def flush():
            ga = pltpu.async_copy(wv_hbm.at[idb], wvg, s0)
            gb = pltpu.async_copy(wd_hbm.at[idbr], wdg, s1)
            ga.wait()
            gb.wait()

            def acc1(i, c2):
                ii = jnp.full((16,), i, jnp.int32)
                rsp = plsc.load_gather(rowb, [ii])
                idsp = plsc.load_gather(idb, [ii])
                dcol = lax.shift_left(idsp & 7, 4) + lanes
                dval = plsc.load_gather(wdg, [ii, dcol])
                plsc.addupdate_scatter(accd, [rsp, lanes], dval)
                for h in range(_H):
                    plsc.addupdate_scatter(accv, [rsp, h * 16 + lanes],
                                           wvg[i, pl.ds(h * 16, 16)])
                return c2
            lax.fori_loop(0, _FB, acc1, 0, unroll=4)
            for t in range(_FB // 16):
                idb[pl.ds(t * 16, 16)] = dummy
                idbr[pl.ds(t * 16, 16)] = dummyr
                rowb[pl.ds(t * 16, 16)] = zi16
            cur[pl.ds(0, 16)] = zi16

        def sblk(sb, c):
            soff = pl.multiple_of(sb * _SCAN, 8)
            pltpu.sync_copy(dst_hbm.at[pl.ds(soff, _SCAN)], dblk)
            for g in range(_SCAN // 16):
                dv = dblk[pl.ds(g * 16, 16)]
                dl = dv - mybase
                m = (dl >= 0) & (dl < _OWN)
                eids = soff + g * 16 + lanes
                cv = cur[pl.ds(0, 16)]
                pos = jnp.maximum(lax.cumsum(m.astype(jnp.int32)) - 1 + cv, 0)
                plsc.store_scatter(idb, [pos], eids, mask=m)
                plsc.store_scatter(idbr, [pos],
                                   lax.shift_right_logical(eids, 3), mask=m)
                plsc.store_scatter(rowb, [pos], dl, mask=m)
                cur[pl.ds(0, 16)] = cv + plsc.all_reduce_population_count(m)

                @pl.when(jnp.any(cur[pl.ds(0, 16)] >= (_FB - 16)))
                def _f():
                    flush()
            return c
        lax.fori_loop(0, _E // _SCAN, sblk, 0)
        flush()

        pltpu.sync_copy(accv, outv_hbm.at[pl.ds(mybase, _OWN)])
        pltpu.sync_copy(accd, outd_hbm.at[pl.ds(mybase, _OWN)])

    return body(wv, wd, dst)


def _post_body(x_ref, accv_ref, accd_ref, skip_ref, exp_ref, a_ref,
               g2_ref, b2_ref, w1_ref, c1_ref, w2_ref, c2_ref, out_ref):
    vsum = accv_ref[...]
    den = accd_ref[...][:, :_H]
    den_exp = jnp.dot(den, exp_ref[...],
                      preferred_element_type=jnp.float32)
    conv = vsum / (den_exp + 1e-16) + skip_ref[...]
    x2 = x_ref[...] + a_ref[0, 0] * conv
    mu = jnp.mean(x2, axis=-1, keepdims=True)
    xc = x2 - mu
    var = jnp.mean(xc * xc, axis=-1, keepdims=True)
    h = xc * jax.lax.rsqrt(var + 1e-5) * g2_ref[...] + b2_ref[...]
    hh = jnp.dot(h, w1_ref[...], preferred_element_type=jnp.float32) + c1_ref[...]
    hh = hh * jax.nn.sigmoid(hh)
    y = jnp.dot(hh, w2_ref[...], preferred_element_type=jnp.float32) + c2_ref[...]
    out_ref[...] = x2 + y


def _post(x, accv, accd, skip, alpha_p, g2, b2, W1, c1, W2, c2):
    grid = _N // _NODE_BLK
    row = pl.BlockSpec((_NODE_BLK, _D), lambda i: (i, 0))
    expand = np.kron(np.eye(_H, dtype=np.float32),
                     np.ones((1, _C), dtype=np.float32))
    return pl.pallas_call(
        _post_body,
        grid=(grid,),
        in_specs=[row,
                  pl.BlockSpec((_NODE_BLK, _D), lambda i: (i, 0)),
                  pl.BlockSpec((_NODE_BLK, 16), lambda i: (i, 0)),
                  row,
                  pl.BlockSpec((_H, _D), lambda i: (0, 0)),
                  pl.BlockSpec((1, 1), lambda i: (0, 0)),
                  pl.BlockSpec((1, _D), lambda i: (0, 0)),
                  pl.BlockSpec((1, _D), lambda i: (0, 0)),
                  pl.BlockSpec((_D, 4 * _D), lambda i: (0, 0)),
                  pl.BlockSpec((1, 4 * _D), lambda i: (0, 0)),
                  pl.BlockSpec((4 * _D, _D), lambda i: (0, 0)),
                  pl.BlockSpec((1, _D), lambda i: (0, 0))],
        out_specs=row,
        out_shape=jax.ShapeDtypeStruct((_N, _D), jnp.float32),
    )(x, accv, accd, skip, jnp.asarray(expand), alpha_p.reshape(1, 1),
      g2.reshape(1, _D), b2.reshape(1, _D),
      W1, c1.reshape(1, 4 * _D), W2, c2.reshape(1, _D))


def kernel(x, edge_index, edge_attr, g1, b1, Wq, bq, Wk, bk, Wv, bv, We, be,
           Wskip, bskip, alpha_p, g2, b2, W1, c1, W2, c2):
    q, k, v, skip = _node_proj(x, g1, b1, Wq, bq, Wk, bk, Wv, bv, Wskip, bskip)
    e = _edge_proj(edge_attr, We, be)
    src = edge_index[0].astype(jnp.int32)
    dst = edge_index[1].astype(jnp.int32)
    wv, wd = _edge_contrib_sc(q, k, v, e, src, dst)
    accv, accd = _segment_reduce_sc(wv, wd, dst)
    return _post(x, accv, accd, skip, alpha_p, g2, b2, W1, c1, W2, c2)


# speedup vs baseline: 1.1578x; 1.1578x over previous
import functools

import jax
import jax.numpy as jnp
import numpy as np
from jax import lax
from jax.experimental import pallas as pl
from jax.experimental.pallas import tpu as pltpu
from jax.experimental.pallas import tpu_sc as plsc

_N = 10000
_E = 320000
_D = 128
_H = 8
_C = _D // _H

_NODE_BLK = 1000
_EDGE_BLK = 4000

_NW = 32
_EPT = _E // _NW
_EB = 128
_EPT_MAIN = 10240
_NBLK_MAIN = _EPT_MAIN // _EB
_NPT = 640
_N_PAD = 16 * _NPT
_DPT = _NPT // 16


def _node_proj_body(x_ref, g1_ref, b1_ref, wq_ref, bq_ref, wk_ref, bk_ref,
                    wv_ref, bv_ref, ws_ref, bs_ref,
                    q_ref, k_ref, v_ref, skip_ref):
    x = x_ref[...]
    mu = jnp.mean(x, axis=-1, keepdims=True)
    xc = x - mu
    var = jnp.mean(xc * xc, axis=-1, keepdims=True)
    h = xc * jax.lax.rsqrt(var + 1e-5) * g1_ref[...] + b1_ref[...]
    q = jnp.dot(h, wq_ref[...], preferred_element_type=jnp.float32) + bq_ref[...]
    q_ref[...] = q * (1.0 / np.sqrt(_C))
    k_ref[...] = jnp.dot(h, wk_ref[...], preferred_element_type=jnp.float32) + bk_ref[...]
    v_ref[...] = jnp.dot(h, wv_ref[...], preferred_element_type=jnp.float32) + bv_ref[...]
    skip_ref[...] = jnp.dot(h, ws_ref[...], preferred_element_type=jnp.float32) + bs_ref[...]


def _node_proj(x, g1, b1, Wq, bq, Wk, bk, Wv, bv, Wskip, bskip):
    grid = _N // _NODE_BLK
    row = pl.BlockSpec((_NODE_BLK, _D), lambda i: (i, 0))
    full = pl.BlockSpec((_D, _D), lambda i: (0, 0))
    vec = pl.BlockSpec((1, _D), lambda i: (0, 0))
    out_sd = jax.ShapeDtypeStruct((_N, _D), jnp.float32)
    return pl.pallas_call(
        _node_proj_body,
        grid=(grid,),
        in_specs=[row, vec, vec, full, vec, full, vec, full, vec, full, vec],
        out_specs=[row, row, row, row],
        out_shape=[out_sd, out_sd, out_sd, out_sd],
    )(x, g1.reshape(1, _D), b1.reshape(1, _D), Wq, bq.reshape(1, _D),
      Wk, bk.reshape(1, _D), Wv, bv.reshape(1, _D), Wskip, bskip.reshape(1, _D))


def _edge_proj_body(ea_ref, we_ref, be_ref, e_ref):
    e_ref[...] = jnp.dot(ea_ref[...], we_ref[...],
                         preferred_element_type=jnp.float32) + be_ref[...]


def _edge_proj(edge_attr, We, be):
    grid = _E // _EDGE_BLK
    row = pl.BlockSpec((_EDGE_BLK, _D), lambda i: (i, 0))
    return pl.pallas_call(
        _edge_proj_body,
        grid=(grid,),
        in_specs=[row, pl.BlockSpec((_D, _D), lambda i: (0, 0)),
                  pl.BlockSpec((1, _D), lambda i: (0, 0))],
        out_specs=row,
        out_shape=jax.ShapeDtypeStruct((_E, _D), jnp.float32),
    )(edge_attr, We, be.reshape(1, _D))


_OWN = _N_PAD // _NW
_SCAN = 512
_FB = 128


def _edge_contrib_sc(q, k, v, e, src, dst):
    mesh = plsc.VectorSubcoreMesh(core_axis_name="c", subcore_axis_name="s")

    @functools.partial(
        pl.kernel, mesh=mesh,
        out_type=[jax.ShapeDtypeStruct((_E + 8, _D), jnp.float32),
                  jax.ShapeDtypeStruct((_E // 8 + 8, _D), jnp.float32)],
        compiler_params=pltpu.CompilerParams(needs_layout_passes=False),
        scratch_types=[
            pltpu.VMEM((_EB,), jnp.int32),
            pltpu.VMEM((_EB,), jnp.int32),
            pltpu.VMEM((_EB, _D), jnp.float32),
            pltpu.VMEM((_EB, _D), jnp.float32),
            pltpu.VMEM((_EB, _D), jnp.float32),
            pltpu.VMEM((_EB, _D), jnp.float32),
            pltpu.VMEM((_EB, _D), jnp.float32),
            pltpu.VMEM((_EB // 8, _D), jnp.float32),
            pltpu.SemaphoreType.DMA,
            pltpu.SemaphoreType.DMA,
            pltpu.SemaphoreType.DMA,
            pltpu.SemaphoreType.DMA,
        ],
    )
    def body(q_hbm, k_hbm, v_hbm, e_hbm, src_hbm, dst_hbm, wv_hbm, wd_hbm,
             si, di, qr, kr, vr, er, orow, orowd, s0, s1, s2, s3):
        cid = lax.axis_index("c")
        sid = lax.axis_index("s")
        wid = sid * 2 + cid
        zeros16 = jnp.zeros((16,), jnp.float32)
        lanes = lax.iota(jnp.int32, 16)

        def zrow(r, c):
            for cb in range(_D // 16):
                orow[r, pl.ds(cb * 16, 16)] = zeros16
            return c
        lax.fori_loop(0, _EB, zrow, 0)

        def zrowd(r, c):
            for cb in range(_D // 16):
                orowd[r, pl.ds(cb * 16, 16)] = zeros16
            return c
        lax.fori_loop(0, _EB // 8, zrowd, 0)

        @pl.when(wid == 0)
        def _zpad():
            pltpu.sync_copy(orow.at[pl.ds(0, 8)], wv_hbm.at[pl.ds(_E, 8)])
            pltpu.sync_copy(orowd.at[pl.ds(0, 8)], wd_hbm.at[pl.ds(_E // 8, 8)])

        ebase = wid * _EPT_MAIN
        nblk = jnp.where(wid < 31, _NBLK_MAIN, (_E - 31 * _EPT_MAIN) // _EB)

        def eblk(j, c):
            off = pl.multiple_of(ebase + j * _EB, 8)
            pltpu.sync_copy(src_hbm.at[pl.ds(off, _EB)], si)
            pltpu.sync_copy(dst_hbm.at[pl.ds(off, _EB)], di)
            cq = pltpu.async_copy(q_hbm.at[di], qr, s0)
            ck = pltpu.async_copy(k_hbm.at[si], kr, s1)
            cv = pltpu.async_copy(v_hbm.at[si], vr, s2)
            ce = pltpu.async_copy(e_hbm.at[pl.ds(off, _EB)], er, s3)
            cq.wait()
            ck.wait()
            cv.wait()
            ce.wait()

            def edge(i, c2):
                dv = zeros16
                for h in range(_H):
                    sl = pl.ds(h * 16, 16)
                    ev = er[i, sl]
                    t = jnp.sum(qr[i, sl] * (kr[i, sl] + ev))
                    w = jnp.exp(jnp.full((16,), t, jnp.float32))
                    orow[i, sl] = w * (vr[i, sl] + ev)
                    dv = jnp.where(lanes == h, w, dv)
                orowd[lax.shift_right_logical(i, 3),
                      pl.ds((i & 7) * 16, 16)] = dv
                return c2
            lax.fori_loop(0, _EB, edge, 0)
            pltpu.sync_copy(orow, wv_hbm.at[pl.ds(off, _EB)])
            pltpu.sync_copy(
                orowd,
                wd_hbm.at[pl.ds(pl.multiple_of(
                    lax.shift_right_logical(off, 3), 8), _EB // 8)])
            return c
        lax.fori_loop(0, nblk, eblk, 0)

    return body(q, k, v, e, src, dst)


def _segment_reduce_sc(wv, wd, dst):
    mesh = plsc.VectorSubcoreMesh(core_axis_name="c", subcore_axis_name="s")

    @functools.partial(
        pl.kernel, mesh=mesh,
        out_type=[jax.ShapeDtypeStruct((_N_PAD, _D), jnp.float32),
                  jax.ShapeDtypeStruct((_N_PAD, 16), jnp.float32)],
        compiler_params=pltpu.CompilerParams(needs_layout_passes=False),
        scratch_types=[
            pltpu.VMEM((_OWN, _D), jnp.float32),
            pltpu.VMEM((_OWN, 16), jnp.float32),
            pltpu.VMEM((_SCAN,), jnp.int32),
            pltpu.VMEM((_FB,), jnp.int32),
            pltpu.VMEM((_FB,), jnp.int32),
            pltpu.VMEM((_FB,), jnp.int32),
            pltpu.VMEM((16,), jnp.int32),
            pltpu.VMEM((_FB, _D), jnp.float32),
            pltpu.VMEM((_FB, _D), jnp.float32),
            pltpu.SemaphoreType.DMA,
            pltpu.SemaphoreType.DMA,
        ],
    )
    def body(wv_hbm, wd_hbm, dst_hbm, outv_hbm, outd_hbm,
             accv, accd, dblk, idb, idbr, rowb, cur, wvg, wdg, s0, s1):
        cid = lax.axis_index("c")
        sid = lax.axis_index("s")
        wid = sid * 2 + cid
        zeros16 = jnp.zeros((16,), jnp.float32)
        zi16 = jnp.zeros((16,), jnp.int32)
        lanes = lax.iota(jnp.int32, 16)
        mybase = wid * _OWN
        dummy = jnp.full((16,), _E, jnp.int32)

        dummyr = jnp.full((16,), _E // 8, jnp.int32)

        def zacc(r, c):
            for cb in range(_D // 16):
                accv[r, pl.ds(cb * 16, 16)] = zeros16
            accd[r, pl.ds(0, 16)] = zeros16
            return c
        lax.fori_loop(0, _OWN, zacc, 0)
        for t in range(_FB // 16):
            idb[pl.ds(t * 16, 16)] = dummy
            idbr[pl.ds(t * 16, 16)] = dummyr
            rowb[pl.ds(t * 16, 16)] = zi16
        cur[pl.ds(0, 16)] = zi16

        def flush():
            ga = pltpu.async_copy(wv_hbm.at[idb], wvg, s0)
            gb = pltpu.async_copy(wd_hbm.at[idbr], wdg, s1)
            ga.wait()
            gb.wait()

            def acc1(i, c2):
                ii = jnp.full((16,), i, jnp.int32)
                rsp = plsc.load_gather(rowb, [ii])
                idsp = plsc.load_gather(idb, [ii])
                dcol = lax.shift_left(idsp & 7, 4) + lanes
                dval = plsc.load_gather(wdg, [ii, dcol])
                plsc.addupdate_scatter(accd, [rsp, lanes], dval)
                for h in range(_H):
                    plsc.addupdate_scatter(accv, [rsp, h * 16 + lanes],
                                           wvg[i, pl.ds(h * 16, 16)])
                return c2
            lax.fori_loop(0, _FB, acc1, 0)
            for t in range(_FB // 16):
                idb[pl.ds(t * 16, 16)] = dummy
                idbr[pl.ds(t * 16, 16)] = dummyr
                rowb[pl.ds(t * 16, 16)] = zi16
            cur[pl.ds(0, 16)] = zi16

        def sblk(sb, c):
            soff = pl.multiple_of(sb * _SCAN, 8)
            pltpu.sync_copy(dst_hbm.at[pl.ds(soff, _SCAN)], dblk)
            for g in range(_SCAN // 16):
                dv = dblk[pl.ds(g * 16, 16)]
                dl = dv - mybase
                m = (dl >= 0) & (dl < _OWN)
                eids = soff + g * 16 + lanes
                cv = cur[pl.ds(0, 16)]
                pos = jnp.maximum(lax.cumsum(m.astype(jnp.int32)) - 1 + cv, 0)
                plsc.store_scatter(idb, [pos], eids, mask=m)
                plsc.store_scatter(idbr, [pos],
                                   lax.shift_right_logical(eids, 3), mask=m)
                plsc.store_scatter(rowb, [pos], dl, mask=m)
                cur[pl.ds(0, 16)] = cv + plsc.all_reduce_population_count(m)

                @pl.when(jnp.any(cur[pl.ds(0, 16)] >= (_FB - 16)))
                def _f():
                    flush()
            return c
        lax.fori_loop(0, _E // _SCAN, sblk, 0)
        flush()

        pltpu.sync_copy(accv, outv_hbm.at[pl.ds(mybase, _OWN)])
        pltpu.sync_copy(accd, outd_hbm.at[pl.ds(mybase, _OWN)])

    return body(wv, wd, dst)


def _post_body(x_ref, accv_ref, accd_ref, skip_ref, exp_ref, a_ref,
               g2_ref, b2_ref, w1_ref, c1_ref, w2_ref, c2_ref, out_ref):
    vsum = accv_ref[...]
    den = accd_ref[...][:, :_H]
    den_exp = jnp.dot(den, exp_ref[...],
                      preferred_element_type=jnp.float32)
    conv = vsum / (den_exp + 1e-16) + skip_ref[...]
    x2 = x_ref[...] + a_ref[0, 0] * conv
    mu = jnp.mean(x2, axis=-1, keepdims=True)
    xc = x2 - mu
    var = jnp.mean(xc * xc, axis=-1, keepdims=True)
    h = xc * jax.lax.rsqrt(var + 1e-5) * g2_ref[...] + b2_ref[...]
    hh = jnp.dot(h, w1_ref[...], preferred_element_type=jnp.float32) + c1_ref[...]
    hh = hh * jax.nn.sigmoid(hh)
    y = jnp.dot(hh, w2_ref[...], preferred_element_type=jnp.float32) + c2_ref[...]
    out_ref[...] = x2 + y


def _post(x, accv, accd, skip, alpha_p, g2, b2, W1, c1, W2, c2):
    grid = _N // _NODE_BLK
    row = pl.BlockSpec((_NODE_BLK, _D), lambda i: (i, 0))
    expand = np.kron(np.eye(_H, dtype=np.float32),
                     np.ones((1, _C), dtype=np.float32))
    return pl.pallas_call(
        _post_body,
        grid=(grid,),
        in_specs=[row,
                  pl.BlockSpec((_NODE_BLK, _D), lambda i: (i, 0)),
                  pl.BlockSpec((_NODE_BLK, 16), lambda i: (i, 0)),
                  row,
                  pl.BlockSpec((_H, _D), lambda i: (0, 0)),
                  pl.BlockSpec((1, 1), lambda i: (0, 0)),
                  pl.BlockSpec((1, _D), lambda i: (0, 0)),
                  pl.BlockSpec((1, _D), lambda i: (0, 0)),
                  pl.BlockSpec((_D, 4 * _D), lambda i: (0, 0)),
                  pl.BlockSpec((1, 4 * _D), lambda i: (0, 0)),
                  pl.BlockSpec((4 * _D, _D), lambda i: (0, 0)),
                  pl.BlockSpec((1, _D), lambda i: (0, 0))],
        out_specs=row,
        out_shape=jax.ShapeDtypeStruct((_N, _D), jnp.float32),
    )(x, accv, accd, skip, jnp.asarray(expand), alpha_p.reshape(1, 1),
      g2.reshape(1, _D), b2.reshape(1, _D),
      W1, c1.reshape(1, 4 * _D), W2, c2.reshape(1, _D))


def kernel(x, edge_index, edge_attr, g1, b1, Wq, bq, Wk, bk, Wv, bv, We, be,
           Wskip, bskip, alpha_p, g2, b2, W1, c1, W2, c2):
    q, k, v, skip = _node_proj(x, g1, b1, Wq, bq, Wk, bk, Wv, bv, Wskip, bskip)
    e = _edge_proj(edge_attr, We, be)
    src = edge_index[0].astype(jnp.int32)
    dst = edge_index[1].astype(jnp.int32)
    wv, wd = _edge_contrib_sc(q, k, v, e, src, dst)
    accv, accd = _segment_reduce_sc(wv, wd, dst)
    return _post(x, accv, accd, skip, alpha_p, g2, b2, W1, c1, W2, c2)


# speedup vs baseline: 1.1718x; 1.0120x over previous
import functools

import jax
import jax.numpy as jnp
import numpy as np
from jax import lax
from jax.experimental import pallas as pl
from jax.experimental.pallas import tpu as pltpu
from jax.experimental.pallas import tpu_sc as plsc

_N = 10000
_E = 320000
_D = 128
_H = 8
_C = _D // _H

_NODE_BLK = 1000
_EDGE_BLK = 4000

_NW = 32
_EPT = _E // _NW
_EB = 128
_EPT_MAIN = 10240
_NBLK_MAIN = _EPT_MAIN // _EB
_NPT = 640
_N_PAD = 16 * _NPT
_DPT = _NPT // 16


def _node_proj_body(x_ref, g1_ref, b1_ref, wq_ref, bq_ref, wk_ref, bk_ref,
                    wv_ref, bv_ref, ws_ref, bs_ref,
                    q_ref, k_ref, v_ref, skip_ref):
    x = x_ref[...]
    mu = jnp.mean(x, axis=-1, keepdims=True)
    xc = x - mu
    var = jnp.mean(xc * xc, axis=-1, keepdims=True)
    h = xc * jax.lax.rsqrt(var + 1e-5) * g1_ref[...] + b1_ref[...]
    q = jnp.dot(h, wq_ref[...], preferred_element_type=jnp.float32) + bq_ref[...]
    q_ref[...] = q * (1.0 / np.sqrt(_C))
    k_ref[...] = jnp.dot(h, wk_ref[...], preferred_element_type=jnp.float32) + bk_ref[...]
    v_ref[...] = jnp.dot(h, wv_ref[...], preferred_element_type=jnp.float32) + bv_ref[...]
    skip_ref[...] = jnp.dot(h, ws_ref[...], preferred_element_type=jnp.float32) + bs_ref[...]


def _node_proj(x, g1, b1, Wq, bq, Wk, bk, Wv, bv, Wskip, bskip):
    grid = _N // _NODE_BLK
    row = pl.BlockSpec((_NODE_BLK, _D), lambda i: (i, 0))
    full = pl.BlockSpec((_D, _D), lambda i: (0, 0))
    vec = pl.BlockSpec((1, _D), lambda i: (0, 0))
    out_sd = jax.ShapeDtypeStruct((_N, _D), jnp.float32)
    return pl.pallas_call(
        _node_proj_body,
        grid=(grid,),
        in_specs=[row, vec, vec, full, vec, full, vec, full, vec, full, vec],
        out_specs=[row, row, row, row],
        out_shape=[out_sd, out_sd, out_sd, out_sd],
    )(x, g1.reshape(1, _D), b1.reshape(1, _D), Wq, bq.reshape(1, _D),
      Wk, bk.reshape(1, _D), Wv, bv.reshape(1, _D), Wskip, bskip.reshape(1, _D))


def _edge_proj_body(ea_ref, we_ref, be_ref, e_ref):
    e_ref[...] = jnp.dot(ea_ref[...], we_ref[...],
                         preferred_element_type=jnp.float32) + be_ref[...]


def _edge_proj(edge_attr, We, be):
    grid = _E // _EDGE_BLK
    row = pl.BlockSpec((_EDGE_BLK, _D), lambda i: (i, 0))
    return pl.pallas_call(
        _edge_proj_body,
        grid=(grid,),
        in_specs=[row, pl.BlockSpec((_D, _D), lambda i: (0, 0)),
                  pl.BlockSpec((1, _D), lambda i: (0, 0))],
        out_specs=row,
        out_shape=jax.ShapeDtypeStruct((_E, _D), jnp.float32),
    )(edge_attr, We, be.reshape(1, _D))


_OWN = _N_PAD // _NW
_SCAN = 512
_FB = 128


def _edge_contrib_sc(q, k, v, e, src, dst):
    mesh = plsc.VectorSubcoreMesh(core_axis_name="c", subcore_axis_name="s")

    @functools.partial(
        pl.kernel, mesh=mesh,
        out_type=[jax.ShapeDtypeStruct((_E + 8, _D), jnp.float32),
                  jax.ShapeDtypeStruct((_E // 8 + 8, _D), jnp.float32)],
        compiler_params=pltpu.CompilerParams(needs_layout_passes=False),
        scratch_types=[
            pltpu.VMEM((_EB,), jnp.int32),
            pltpu.VMEM((_EB,), jnp.int32),
            pltpu.VMEM((_EB, _D), jnp.float32),
            pltpu.VMEM((_EB, _D), jnp.float32),
            pltpu.VMEM((_EB, _D), jnp.float32),
            pltpu.VMEM((_EB, _D), jnp.float32),
            pltpu.VMEM((_EB, _D), jnp.float32),
            pltpu.VMEM((_EB // 8, _D), jnp.float32),
            pltpu.SemaphoreType.DMA,
            pltpu.SemaphoreType.DMA,
            pltpu.SemaphoreType.DMA,
            pltpu.SemaphoreType.DMA,
            pltpu.SemaphoreType.DMA,
            pltpu.SemaphoreType.DMA,
        ],
    )
    def body(q_hbm, k_hbm, v_hbm, e_hbm, src_hbm, dst_hbm, wv_hbm, wd_hbm,
             si, di, qr, kr, vr, er, orow, orowd, s0, s1, s2, s3, s4, s5):
        cid = lax.axis_index("c")
        sid = lax.axis_index("s")
        wid = sid * 2 + cid
        zeros16 = jnp.zeros((16,), jnp.float32)
        lanes = lax.iota(jnp.int32, 16)

        def zrow(r, c):
            for cb in range(_D // 16):
                orow[r, pl.ds(cb * 16, 16)] = zeros16
            return c
        lax.fori_loop(0, _EB, zrow, 0)

        def zrowd(r, c):
            for cb in range(_D // 16):
                orowd[r, pl.ds(cb * 16, 16)] = zeros16
            return c
        lax.fori_loop(0, _EB // 8, zrowd, 0)

        @pl.when(wid == 0)
        def _zpad():
            pltpu.sync_copy(orow.at[pl.ds(0, 8)], wv_hbm.at[pl.ds(_E, 8)])
            pltpu.sync_copy(orowd.at[pl.ds(0, 8)], wd_hbm.at[pl.ds(_E // 8, 8)])

        ebase = wid * _EPT_MAIN
        nblk = jnp.where(wid < 31, _NBLK_MAIN, (_E - 31 * _EPT_MAIN) // _EB)

        def eblk(j, c):
            off = pl.multiple_of(ebase + j * _EB, 8)
            ce = pltpu.async_copy(e_hbm.at[pl.ds(off, _EB)], er, s3)
            cs = pltpu.async_copy(src_hbm.at[pl.ds(off, _EB)], si, s4)
            cd = pltpu.async_copy(dst_hbm.at[pl.ds(off, _EB)], di, s5)
            cs.wait()
            cd.wait()
            cq = pltpu.async_copy(q_hbm.at[di], qr, s0)
            ck = pltpu.async_copy(k_hbm.at[si], kr, s1)
            cv = pltpu.async_copy(v_hbm.at[si], vr, s2)
            cq.wait()
            ck.wait()
            cv.wait()
            ce.wait()

            def edge(i, c2):
                dv = zeros16
                for h in range(_H):
                    sl = pl.ds(h * 16, 16)
                    ev = er[i, sl]
                    t = jnp.sum(qr[i, sl] * (kr[i, sl] + ev))
                    w = jnp.exp(jnp.full((16,), t, jnp.float32))
                    orow[i, sl] = w * (vr[i, sl] + ev)
                    dv = jnp.where(lanes == h, w, dv)
                orowd[lax.shift_right_logical(i, 3),
                      pl.ds((i & 7) * 16, 16)] = dv
                return c2
            lax.fori_loop(0, _EB, edge, 0)
            pltpu.sync_copy(orow, wv_hbm.at[pl.ds(off, _EB)])
            pltpu.sync_copy(
                orowd,
                wd_hbm.at[pl.ds(pl.multiple_of(
                    lax.shift_right_logical(off, 3), 8), _EB // 8)])
            return c
        lax.fori_loop(0, nblk, eblk, 0)

    return body(q, k, v, e, src, dst)


def _segment_reduce_sc(wv, wd, dst):
    mesh = plsc.VectorSubcoreMesh(core_axis_name="c", subcore_axis_name="s")

    @functools.partial(
        pl.kernel, mesh=mesh,
        out_type=[jax.ShapeDtypeStruct((_N_PAD, _D), jnp.float32),
                  jax.ShapeDtypeStruct((_N_PAD, 16), jnp.float32)],
        compiler_params=pltpu.CompilerParams(needs_layout_passes=False),
        scratch_types=[
            pltpu.VMEM((_OWN, _D), jnp.float32),
            pltpu.VMEM((_OWN, 16), jnp.float32),
            pltpu.VMEM((_SCAN,), jnp.int32),
            pltpu.VMEM((_FB,), jnp.int32),
            pltpu.VMEM((_FB,), jnp.int32),
            pltpu.VMEM((_FB,), jnp.int32),
            pltpu.VMEM((16,), jnp.int32),
            pltpu.VMEM((_FB, _D), jnp.float32),
            pltpu.VMEM((_FB, _D), jnp.float32),
            pltpu.SemaphoreType.DMA,
            pltpu.SemaphoreType.DMA,
        ],
    )
    def body(wv_hbm, wd_hbm, dst_hbm, outv_hbm, outd_hbm,
             accv, accd, dblk, idb, idbr, rowb, cur, wvg, wdg, s0, s1):
        cid = lax.axis_index("c")
        sid = lax.axis_index("s")
        wid = sid * 2 + cid
        zeros16 = jnp.zeros((16,), jnp.float32)
        zi16 = jnp.zeros((16,), jnp.int32)
        lanes = lax.iota(jnp.int32, 16)
        mybase = wid * _OWN
        dummy = jnp.full((16,), _E, jnp.int32)

        dummyr = jnp.full((16,), _E // 8, jnp.int32)

        def zacc(r, c):
            for cb in range(_D // 16):
                accv[r, pl.ds(cb * 16, 16)] = zeros16
            accd[r, pl.ds(0, 16)] = zeros16
            return c
        lax.fori_loop(0, _OWN, zacc, 0)
        for t in range(_FB // 16):
            idb[pl.ds(t * 16, 16)] = dummy
            idbr[pl.ds(t * 16, 16)] = dummyr
            rowb[pl.ds(t * 16, 16)] = zi16
        cur[pl.ds(0, 16)] = zi16

        def flush():
            ga = pltpu.async_copy(wv_hbm.at[idb], wvg, s0)
            gb = pltpu.async_copy(wd_hbm.at[idbr], wdg, s1)
            ga.wait()
            gb.wait()

            def acc1(i, c2):
                ii = jnp.full((16,), i, jnp.int32)
                rsp = plsc.load_gather(rowb, [ii])
                idsp = plsc.load_gather(idb, [ii])
                dcol = lax.shift_left(idsp & 7, 4) + lanes
                dval = plsc.load_gather(wdg, [ii, dcol])
                plsc.addupdate_scatter(accd, [rsp, lanes], dval)
                for h in range(_H):
                    plsc.addupdate_scatter(accv, [rsp, h * 16 + lanes],
                                           wvg[i, pl.ds(h * 16, 16)])
                return c2
            lax.fori_loop(0, _FB, acc1, 0)
            for t in range(_FB // 16):
                idb[pl.ds(t * 16, 16)] = dummy
                idbr[pl.ds(t * 16, 16)] = dummyr
                rowb[pl.ds(t * 16, 16)] = zi16
            cur[pl.ds(0, 16)] = zi16

        def sblk(sb, c):
            soff = pl.multiple_of(sb * _SCAN, 8)
            pltpu.sync_copy(dst_hbm.at[pl.ds(soff, _SCAN)], dblk)
            for g in range(_SCAN // 16):
                dv = dblk[pl.ds(g * 16, 16)]
                dl = dv - mybase
                m = (dl >= 0) & (dl < _OWN)
                eids = soff + g * 16 + lanes
                cv = cur[pl.ds(0, 16)]
                pos = jnp.maximum(lax.cumsum(m.astype(jnp.int32)) - 1 + cv, 0)
                plsc.store_scatter(idb, [pos], eids, mask=m)
                plsc.store_scatter(idbr, [pos],
                                   lax.shift_right_logical(eids, 3), mask=m)
                plsc.store_scatter(rowb, [pos], dl, mask=m)
                cur[pl.ds(0, 16)] = cv + plsc.all_reduce_population_count(m)

                @pl.when(jnp.any(cur[pl.ds(0, 16)] >= (_FB - 16)))
                def _f():
                    flush()
            return c
        lax.fori_loop(0, _E // _SCAN, sblk, 0)
        flush()

        pltpu.sync_copy(accv, outv_hbm.at[pl.ds(mybase, _OWN)])
        pltpu.sync_copy(accd, outd_hbm.at[pl.ds(mybase, _OWN)])

    return body(wv, wd, dst)


def _post_body(x_ref, accv_ref, accd_ref, skip_ref, exp_ref, a_ref,
               g2_ref, b2_ref, w1_ref, c1_ref, w2_ref, c2_ref, out_ref):
    vsum = accv_ref[...]
    den = accd_ref[...][:, :_H]
    den_exp = jnp.dot(den, exp_ref[...],
                      preferred_element_type=jnp.float32)
    conv = vsum / (den_exp + 1e-16) + skip_ref[...]
    x2 = x_ref[...] + a_ref[0, 0] * conv
    mu = jnp.mean(x2, axis=-1, keepdims=True)
    xc = x2 - mu
    var = jnp.mean(xc * xc, axis=-1, keepdims=True)
    h = xc * jax.lax.rsqrt(var + 1e-5) * g2_ref[...] + b2_ref[...]
    hh = jnp.dot(h, w1_ref[...], preferred_element_type=jnp.float32) + c1_ref[...]
    hh = hh * jax.nn.sigmoid(hh)
    y = jnp.dot(hh, w2_ref[...], preferred_element_type=jnp.float32) + c2_ref[...]
    out_ref[...] = x2 + y


def _post(x, accv, accd, skip, alpha_p, g2, b2, W1, c1, W2, c2):
    grid = _N // _NODE_BLK
    row = pl.BlockSpec((_NODE_BLK, _D), lambda i: (i, 0))
    expand = np.kron(np.eye(_H, dtype=np.float32),
                     np.ones((1, _C), dtype=np.float32))
    return pl.pallas_call(
        _post_body,
        grid=(grid,),
        in_specs=[row,
                  pl.BlockSpec((_NODE_BLK, _D), lambda i: (i, 0)),
                  pl.BlockSpec((_NODE_BLK, 16), lambda i: (i, 0)),
                  row,
                  pl.BlockSpec((_H, _D), lambda i: (0, 0)),
                  pl.BlockSpec((1, 1), lambda i: (0, 0)),
                  pl.BlockSpec((1, _D), lambda i: (0, 0)),
                  pl.BlockSpec((1, _D), lambda i: (0, 0)),
                  pl.BlockSpec((_D, 4 * _D), lambda i: (0, 0)),
                  pl.BlockSpec((1, 4 * _D), lambda i: (0, 0)),
                  pl.BlockSpec((4 * _D, _D), lambda i: (0, 0)),
                  pl.BlockSpec((1, _D), lambda i: (0, 0))],
        out_specs=row,
        out_shape=jax.ShapeDtypeStruct((_N, _D), jnp.float32),
    )(x, accv, accd, skip, jnp.asarray(expand), alpha_p.reshape(1, 1),
      g2.reshape(1, _D), b2.reshape(1, _D),
      W1, c1.reshape(1, 4 * _D), W2, c2.reshape(1, _D))


def kernel(x, edge_index, edge_attr, g1, b1, Wq, bq, Wk, bk, Wv, bv, We, be,
           Wskip, bskip, alpha_p, g2, b2, W1, c1, W2, c2):
    q, k, v, skip = _node_proj(x, g1, b1, Wq, bq, Wk, bk, Wv, bv, Wskip, bskip)
    e = _edge_proj(edge_attr, We, be)
    src = edge_index[0].astype(jnp.int32)
    dst = edge_index[1].astype(jnp.int32)
    wv, wd = _edge_contrib_sc(q, k, v, e, src, dst)
    accv, accd = _segment_reduce_sc(wv, wd, dst)
    return _post(x, accv, accd, skip, alpha_p, g2, b2, W1, c1, W2, c2)
